# baseline (speedup 1.0000x reference)
import functools

import jax
import jax.numpy as jnp
from jax import lax
from jax.experimental import pallas as pl
from jax.experimental.pallas import tpu as pltpu
from jax.experimental.pallas import tpu_sc as plsc

N, E, D, P = 10000, 320000, 128, 10000
NC, NS = 2, 16
NW = NC * NS
CHUNK = 64
CPT = 160
C0, C1 = 320, 0
SSTAGE = 32
E_PAD = NW * CPT * CHUNK
ROWS_PER_TILE = 632
N_ACC = ROWS_PER_TILE * NS
LAST_BASE = (NS - 1) * ROWS_PER_TILE
LAST_ROWS = N - LAST_BASE

PCHUNK = 40
PCPT = 8
P_PAD = NW * PCPT * PCHUNK
PAIR_ROWS = 4 * NW * PCPT

@functools.lru_cache(maxsize=None)
def _sc_mesh():
    return plsc.VectorSubcoreMesh(
        core_axis_name="c", subcore_axis_name="s", num_cores=NC, num_subcores=NS
    )


NBUF = 4


def _seg_sum_body(h_hbm, src_hbm, dst_hbm, zero_hbm, out_hbm,
                  idx_src, idx_dst, rows, gsem, ssem, acc):
    cid = lax.axis_index("c")
    sid = lax.axis_index("s")
    wid = sid * NC + cid

    @pl.when(cid == 0)
    def _():
        pltpu.sync_copy(zero_hbm,
                        acc.at[pl.ds(sid * ROWS_PER_TILE, ROWS_PER_TILE)])

    plsc.subcore_barrier()

    base_chunk = jnp.where(cid == 0, sid * C0, NS * C0 + sid * C1)
    nstages = jnp.where(cid == 0, C0 // SSTAGE, C1 // SSTAGE)

    def stage_body(s, carry):
        base = base_chunk + s * SSTAGE
        pltpu.sync_copy(src_hbm.at[pl.ds(base, SSTAGE)], idx_src)
        pltpu.sync_copy(dst_hbm.at[pl.ds(base, SSTAGE)], idx_dst)
        for b in range(NBUF):
            pltpu.async_copy(h_hbm.at[idx_src.at[b]], rows.at[b], gsem)

        def outer(k, carry2):
            g = k * NBUF
            for b in range(NBUF):
                j = g + b
                pltpu.make_async_copy(
                    h_hbm.at[idx_src.at[j]], rows.at[b], gsem).wait()
                pltpu.async_copy(rows.at[b], acc.at[idx_dst.at[j]], ssem,
                                 add=True)
            for b in range(NBUF):
                j = g + b
                pltpu.make_async_copy(
                    rows.at[b], acc.at[idx_dst.at[j]], ssem).wait()

                @pl.when(j + NBUF < SSTAGE)
                def _():
                    pltpu.async_copy(
                        h_hbm.at[idx_src.at[j + NBUF]], rows.at[b], gsem)
            return carry2

        lax.fori_loop(0, SSTAGE // NBUF, outer, 0)
        return carry

    lax.fori_loop(0, nstages, stage_body, 0)
    plsc.subcore_barrier()

    @pl.when((cid == 0) & (sid < NS - 1))
    def _():
        base = sid * ROWS_PER_TILE
        pltpu.sync_copy(acc.at[pl.ds(base, ROWS_PER_TILE)],
                        out_hbm.at[pl.ds(base, ROWS_PER_TILE)])

    @pl.when((cid == 0) & (sid == NS - 1))
    def _():
        pltpu.sync_copy(acc.at[pl.ds(LAST_BASE, LAST_ROWS)],
                        out_hbm.at[pl.ds(LAST_BASE, LAST_ROWS)])


@functools.lru_cache(maxsize=None)
def _seg_sum():
    return pl.kernel(
        _seg_sum_body,
        out_type=jax.ShapeDtypeStruct((N, D), jnp.float32),
        mesh=_sc_mesh(),
        name="seg_sum",
        scratch_types=[
            pltpu.VMEM((SSTAGE, CHUNK), jnp.int32),
            pltpu.VMEM((SSTAGE, CHUNK), jnp.int32),
            pltpu.VMEM((NBUF, CHUNK, D), jnp.float32),
            pltpu.SemaphoreType.DMA,
            pltpu.SemaphoreType.DMA,
            pltpu.VMEM_SHARED((N_ACC, D), jnp.float32),
        ],
    )


NT0, NT1 = 6, 2


def _pair_gather_body(h_hbm, idx_hbm, out_hbm, idxb, rows, gsem, wsem):
    cid = lax.axis_index("c")
    sid = lax.axis_index("s")

    tbase = jnp.where(cid == 0, sid * NT0, NS * NT0 + sid * NT1)
    ntasks = jnp.where(cid == 0, NT0, NT1)

    def task_body(ti, carry):
        t = tbase + ti
        row0 = t * PCPT
        pltpu.sync_copy(idx_hbm.at[pl.ds(row0, PCPT)], idxb)
        for b in range(NBUF):
            pltpu.async_copy(h_hbm.at[idxb.at[b]], rows.at[b], gsem)

        def outer(k, carry2):
            g = k * NBUF
            for b in range(NBUF):
                j = g + b
                pltpu.make_async_copy(
                    h_hbm.at[idxb.at[j]], rows.at[b], gsem).wait()
                pltpu.async_copy(
                    rows.at[b],
                    out_hbm.at[pl.ds((row0 + j) * PCHUNK, PCHUNK)], wsem)
            for b in range(NBUF):
                j = g + b
                pltpu.make_async_copy(
                    rows.at[b],
                    out_hbm.at[pl.ds((row0 + j) * PCHUNK, PCHUNK)], wsem).wait()

                @pl.when(j + NBUF < PCPT)
                def _():
                    pltpu.async_copy(h_hbm.at[idxb.at[j + NBUF]], rows.at[b],
                                     gsem)
            return carry2

        lax.fori_loop(0, PCPT // NBUF, outer, 0)
        return carry

    lax.fori_loop(0, ntasks, task_body, 0)


@functools.lru_cache(maxsize=None)
def _pair_gather():
    return pl.kernel(
        _pair_gather_body,
        out_type=jax.ShapeDtypeStruct((4 * P_PAD, D), jnp.float32),
        mesh=_sc_mesh(),
        scratch_types=[
            pltpu.VMEM((PCPT, PCHUNK), jnp.int32),
            pltpu.VMEM((NBUF, PCHUNK, D), jnp.float32),
            pltpu.SemaphoreType.DMA,
            pltpu.SemaphoreType.DMA,
        ],
    )


def _mlp_body(h_ref, agg_ref, w1_ref, w2_ref, o_ref, *, last):
    x = h_ref[...] + agg_ref[...]
    y = jnp.maximum(jnp.dot(x, w1_ref[...], preferred_element_type=jnp.float32), 0.0)
    z = jnp.dot(y, w2_ref[...], preferred_element_type=jnp.float32)
    if not last:
        z = jnp.maximum(z, 0.0)
    o_ref[...] = z


def _mlp(h, agg, w1, w2, last):
    bn = 1000
    return pl.pallas_call(
        functools.partial(_mlp_body, last=last),
        grid=(N // bn,),
        in_specs=[
            pl.BlockSpec((bn, D), lambda i: (i, 0)),
            pl.BlockSpec((bn, D), lambda i: (i, 0)),
            pl.BlockSpec((D, D), lambda i: (0, 0)),
            pl.BlockSpec((D, D), lambda i: (0, 0)),
        ],
        out_specs=pl.BlockSpec((bn, D), lambda i: (i, 0)),
        out_shape=jax.ShapeDtypeStruct((N, D), jnp.float32),
    )(h, agg, w1, w2)


def _pred_body(ga_ref, gb_ref, p1_ref, b1_ref, p2_ref, b2_ref, p3_ref, b3_ref,
               o_ref):
    z = ga_ref[0] * gb_ref[0]
    z = jnp.maximum(
        jnp.dot(z, p1_ref[...], preferred_element_type=jnp.float32) + b1_ref[...],
        0.0)
    z = jnp.maximum(
        jnp.dot(z, p2_ref[...], preferred_element_type=jnp.float32) + b2_ref[...],
        0.0)
    o_ref[0] = jnp.dot(z, p3_ref[...], preferred_element_type=jnp.float32) + b3_ref[0, 0]


def _predictor(g, p1, b1, p2, b2, p3, b3):
    bn = 1024
    return pl.pallas_call(
        _pred_body,
        grid=(2, P_PAD // bn),
        in_specs=[
            pl.BlockSpec((1, bn, D), lambda p, i: (2 * p, i, 0)),
            pl.BlockSpec((1, bn, D), lambda p, i: (2 * p + 1, i, 0)),
            pl.BlockSpec((D, D), lambda p, i: (0, 0)),
            pl.BlockSpec((1, D), lambda p, i: (0, 0)),
            pl.BlockSpec((D, D), lambda p, i: (0, 0)),
            pl.BlockSpec((1, D), lambda p, i: (0, 0)),
            pl.BlockSpec((D, 1), lambda p, i: (0, 0)),
            pl.BlockSpec((1, 1), lambda p, i: (0, 0)),
        ],
        out_specs=pl.BlockSpec((1, bn, 1), lambda p, i: (p, i, 0)),
        out_shape=jax.ShapeDtypeStruct((2, P_PAD, 1), jnp.float32),
    )(g, g, p1, b1, p2, b2, p3, b3)


def kernel(x, edge_index, pos_edge_index, neg_edge_index,
           W1_0, W2_0, W1_1, W2_1, W1_2, W2_2,
           P1, b1, P2, b2, P3, b3):
    src = jnp.concatenate(
        [edge_index[0], jnp.zeros((E_PAD - E,), jnp.int32)]).reshape(NW * CPT, CHUNK)
    dst = jnp.concatenate(
        [edge_index[1], jnp.full((E_PAD - E,), N, jnp.int32)]).reshape(NW * CPT, CHUNK)
    zeros = jnp.zeros((ROWS_PER_TILE, D), jnp.float32)

    def pad_idx(a):
        return jnp.concatenate([a, jnp.zeros((P_PAD - P,), jnp.int32)])

    pidx = jnp.concatenate([
        pad_idx(pos_edge_index[0]), pad_idx(pos_edge_index[1]),
        pad_idx(neg_edge_index[0]), pad_idx(neg_edge_index[1]),
    ]).reshape(PAIR_ROWS, PCHUNK)

    h = x
    for l, (w1, w2) in enumerate(((W1_0, W2_0), (W1_1, W2_1), (W1_2, W2_2))):
        agg = _seg_sum()(h, src, dst, zeros)
        h = _mlp(h, agg, w1, w2, last=(l == 2))

    g = _pair_gather()(h, pidx).reshape(4, P_PAD, D)
    out = _predictor(g, P1, b1.reshape(1, D), P2, b2.reshape(1, D),
                     P3, b3.reshape(1, 1))
    return out[0, :P], out[1, :P]


# speedup vs baseline: 1.4857x; 1.4857x over previous
import functools

import jax
import jax.numpy as jnp
from jax import lax
from jax.experimental import pallas as pl
from jax.experimental.pallas import tpu as pltpu
from jax.experimental.pallas import tpu_sc as plsc

N, E, D, P = 10000, 320000, 128, 10000
NC, NS = 2, 16
NW = NC * NS
CHUNK = 64
CPT = 160
C0, C1 = 288, 32
SSTAGE = 32
E_PAD = NW * CPT * CHUNK
ROWS_PER_TILE = 632
N_ACC = ROWS_PER_TILE * NS
LAST_BASE = (NS - 1) * ROWS_PER_TILE
LAST_ROWS = N - LAST_BASE

PCHUNK = 40
PCPT = 8
P_PAD = NW * PCPT * PCHUNK
PAIR_ROWS = 4 * NW * PCPT

@functools.lru_cache(maxsize=None)
def _sc_mesh():
    return plsc.VectorSubcoreMesh(
        core_axis_name="c", subcore_axis_name="s", num_cores=NC, num_subcores=NS
    )


NBUF = 4


def _seg_sum_body(h_hbm, src_hbm, dst_hbm, zero_hbm, out_hbm,
                  idx_src, idx_dst, rows, gsem, ssem, acc):
    cid = lax.axis_index("c")
    sid = lax.axis_index("s")
    wid = sid * NC + cid

    pltpu.sync_copy(zero_hbm, acc.at[pl.ds(sid * ROWS_PER_TILE, ROWS_PER_TILE)])
    plsc.subcore_barrier()

    base_chunk = jnp.where(cid == 0, sid * C0, NS * C0 + sid * C1)
    nstages = jnp.where(cid == 0, C0 // SSTAGE, C1 // SSTAGE)

    def stage_body(s, carry):
        base = base_chunk + s * SSTAGE
        pltpu.sync_copy(src_hbm.at[pl.ds(base, SSTAGE)], idx_src)
        pltpu.sync_copy(dst_hbm.at[pl.ds(base, SSTAGE)], idx_dst)
        for b in range(NBUF):
            pltpu.async_copy(h_hbm.at[idx_src.at[b]], rows.at[b], gsem)

        def outer(k, carry2):
            g = k * NBUF
            for b in range(NBUF):
                j = g + b
                pltpu.make_async_copy(
                    h_hbm.at[idx_src.at[j]], rows.at[b], gsem).wait()
                pltpu.async_copy(rows.at[b], acc.at[idx_dst.at[j]], ssem,
                                 add=True)
            for b in range(NBUF):
                j = g + b
                pltpu.make_async_copy(
                    rows.at[b], acc.at[idx_dst.at[j]], ssem).wait()

                @pl.when(j + NBUF < SSTAGE)
                def _():
                    pltpu.async_copy(
                        h_hbm.at[idx_src.at[j + NBUF]], rows.at[b], gsem)
            return carry2

        lax.fori_loop(0, SSTAGE // NBUF, outer, 0)
        return carry

    lax.fori_loop(0, nstages, stage_body, 0)
    plsc.subcore_barrier()

    @pl.when(sid < NS - 1)
    def _():
        base = sid * ROWS_PER_TILE
        pltpu.sync_copy(acc.at[pl.ds(base, ROWS_PER_TILE)],
                        out_hbm.at[pl.ds(cid * N + base, ROWS_PER_TILE)])

    @pl.when(sid == NS - 1)
    def _():
        pltpu.sync_copy(acc.at[pl.ds(LAST_BASE, LAST_ROWS)],
                        out_hbm.at[pl.ds(cid * N + LAST_BASE, LAST_ROWS)])


@functools.lru_cache(maxsize=None)
def _seg_sum():
    return pl.kernel(
        _seg_sum_body,
        out_type=jax.ShapeDtypeStruct((NC * N, D), jnp.float32),
        mesh=_sc_mesh(),
        name="seg_sum",
        scratch_types=[
            pltpu.VMEM((SSTAGE, CHUNK), jnp.int32),
            pltpu.VMEM((SSTAGE, CHUNK), jnp.int32),
            pltpu.VMEM((NBUF, CHUNK, D), jnp.float32),
            pltpu.SemaphoreType.DMA,
            pltpu.SemaphoreType.DMA,
            pltpu.VMEM_SHARED((N_ACC, D), jnp.float32),
        ],
    )


NT0, NT1 = 6, 2


def _pair_gather_body(h_hbm, idx_hbm, out_hbm, idxb, rows, gsem, wsem):
    cid = lax.axis_index("c")
    sid = lax.axis_index("s")

    tbase = jnp.where(cid == 0, sid * NT0, NS * NT0 + sid * NT1)
    ntasks = jnp.where(cid == 0, NT0, NT1)

    def task_body(ti, carry):
        t = tbase + ti
        row0 = t * PCPT
        pltpu.sync_copy(idx_hbm.at[pl.ds(row0, PCPT)], idxb)
        for b in range(NBUF):
            pltpu.async_copy(h_hbm.at[idxb.at[b]], rows.at[b], gsem)

        def outer(k, carry2):
            g = k * NBUF
            for b in range(NBUF):
                j = g + b
                pltpu.make_async_copy(
                    h_hbm.at[idxb.at[j]], rows.at[b], gsem).wait()
                pltpu.async_copy(
                    rows.at[b],
                    out_hbm.at[pl.ds((row0 + j) * PCHUNK, PCHUNK)], wsem)
            for b in range(NBUF):
                j = g + b
                pltpu.make_async_copy(
                    rows.at[b],
                    out_hbm.at[pl.ds((row0 + j) * PCHUNK, PCHUNK)], wsem).wait()

                @pl.when(j + NBUF < PCPT)
                def _():
                    pltpu.async_copy(h_hbm.at[idxb.at[j + NBUF]], rows.at[b],
                                     gsem)
            return carry2

        lax.fori_loop(0, PCPT // NBUF, outer, 0)
        return carry

    lax.fori_loop(0, ntasks, task_body, 0)


@functools.lru_cache(maxsize=None)
def _pair_gather():
    return pl.kernel(
        _pair_gather_body,
        out_type=jax.ShapeDtypeStruct((4 * P_PAD, D), jnp.float32),
        mesh=_sc_mesh(),
        scratch_types=[
            pltpu.VMEM((PCPT, PCHUNK), jnp.int32),
            pltpu.VMEM((NBUF, PCHUNK, D), jnp.float32),
            pltpu.SemaphoreType.DMA,
            pltpu.SemaphoreType.DMA,
        ],
    )


def _mlp_body(h_ref, agg_ref, w1_ref, w2_ref, o_ref, *, last):
    x = h_ref[...] + agg_ref[0] + agg_ref[1]
    y = jnp.maximum(jnp.dot(x, w1_ref[...], preferred_element_type=jnp.float32), 0.0)
    z = jnp.dot(y, w2_ref[...], preferred_element_type=jnp.float32)
    if not last:
        z = jnp.maximum(z, 0.0)
    o_ref[...] = z


def _mlp(h, agg, w1, w2, last):
    bn = 1000
    return pl.pallas_call(
        functools.partial(_mlp_body, last=last),
        grid=(N // bn,),
        in_specs=[
            pl.BlockSpec((bn, D), lambda i: (i, 0)),
            pl.BlockSpec((NC, bn, D), lambda i: (0, i, 0)),
            pl.BlockSpec((D, D), lambda i: (0, 0)),
            pl.BlockSpec((D, D), lambda i: (0, 0)),
        ],
        out_specs=pl.BlockSpec((bn, D), lambda i: (i, 0)),
        out_shape=jax.ShapeDtypeStruct((N, D), jnp.float32),
    )(h, agg, w1, w2)


def _pred_body(ga_ref, gb_ref, p1_ref, b1_ref, p2_ref, b2_ref, p3_ref, b3_ref,
               o_ref):
    z = ga_ref[0] * gb_ref[0]
    z = jnp.maximum(
        jnp.dot(z, p1_ref[...], preferred_element_type=jnp.float32) + b1_ref[...],
        0.0)
    z = jnp.maximum(
        jnp.dot(z, p2_ref[...], preferred_element_type=jnp.float32) + b2_ref[...],
        0.0)
    o_ref[0] = jnp.dot(z, p3_ref[...], preferred_element_type=jnp.float32) + b3_ref[0, 0]


def _predictor(g, p1, b1, p2, b2, p3, b3):
    bn = 1024
    return pl.pallas_call(
        _pred_body,
        grid=(2, P_PAD // bn),
        in_specs=[
            pl.BlockSpec((1, bn, D), lambda p, i: (2 * p, i, 0)),
            pl.BlockSpec((1, bn, D), lambda p, i: (2 * p + 1, i, 0)),
            pl.BlockSpec((D, D), lambda p, i: (0, 0)),
            pl.BlockSpec((1, D), lambda p, i: (0, 0)),
            pl.BlockSpec((D, D), lambda p, i: (0, 0)),
            pl.BlockSpec((1, D), lambda p, i: (0, 0)),
            pl.BlockSpec((D, 1), lambda p, i: (0, 0)),
            pl.BlockSpec((1, 1), lambda p, i: (0, 0)),
        ],
        out_specs=pl.BlockSpec((1, bn, 1), lambda p, i: (p, i, 0)),
        out_shape=jax.ShapeDtypeStruct((2, P_PAD, 1), jnp.float32),
    )(g, g, p1, b1, p2, b2, p3, b3)


def kernel(x, edge_index, pos_edge_index, neg_edge_index,
           W1_0, W2_0, W1_1, W2_1, W1_2, W2_2,
           P1, b1, P2, b2, P3, b3):
    src = jnp.concatenate(
        [edge_index[0], jnp.zeros((E_PAD - E,), jnp.int32)]).reshape(NW * CPT, CHUNK)
    dst = jnp.concatenate(
        [edge_index[1], jnp.full((E_PAD - E,), N, jnp.int32)]).reshape(NW * CPT, CHUNK)
    zeros = jnp.zeros((ROWS_PER_TILE, D), jnp.float32)

    def pad_idx(a):
        return jnp.concatenate([a, jnp.zeros((P_PAD - P,), jnp.int32)])

    pidx = jnp.concatenate([
        pad_idx(pos_edge_index[0]), pad_idx(pos_edge_index[1]),
        pad_idx(neg_edge_index[0]), pad_idx(neg_edge_index[1]),
    ]).reshape(PAIR_ROWS, PCHUNK)

    h = x
    for l, (w1, w2) in enumerate(((W1_0, W2_0), (W1_1, W2_1), (W1_2, W2_2))):
        agg = _seg_sum()(h, src, dst, zeros).reshape(NC, N, D)
        h = _mlp(h, agg, w1, w2, last=(l == 2))

    g = _pair_gather()(h, pidx).reshape(4, P_PAD, D)
    out = _predictor(g, P1, b1.reshape(1, D), P2, b2.reshape(1, D),
                     P3, b3.reshape(1, 1))
    return out[0, :P], out[1, :P]
